# baseline (speedup 1.0000x reference)
import functools

import jax
import jax.numpy as jnp
from jax.experimental import pallas as pl
from jax.experimental.pallas import tpu as pltpu


def _emb_ln_kernel(scal_ref, pos_ref, kf_ref, w_ref, b_ref, x_ref, o_ref,
                   *, s_len, bblk, chunk):
    n_past = scal_ref[0]
    n_trans = scal_ref[2]
    n_position = n_past + scal_ref[1] + n_trans

    d = pos_ref.shape[1]
    s = jax.lax.broadcasted_iota(jnp.int32, (s_len, d), 0)
    in_trans = (s >= n_past) & (s < n_past + n_trans)
    beyond = s >= n_position
    kf_vec = jnp.where(beyond, kf_ref[2:3, :],
                       jnp.where(in_trans, kf_ref[1:2, :], kf_ref[0:1, :]))
    add = (pos_ref[...] + kf_vec)[None, :, :]
    w = w_ref[...]
    b = b_ref[...]
    inv_d = jnp.float32(1.0 / d)

    def body(i, carry):
        off = i * chunk
        emb = x_ref[pl.ds(off, chunk), :, :] + add
        mean = jnp.sum(emb, axis=-1, keepdims=True) * inv_d
        msq = jnp.sum(emb * emb, axis=-1, keepdims=True) * inv_d
        var = msq - mean * mean
        scale = jax.lax.rsqrt(var + 1e-5)
        o_ref[pl.ds(off, chunk), :, :] = (emb - mean) * (scale * w) + b
        return carry

    jax.lax.fori_loop(0, bblk // chunk, body, 0, unroll=False)


def kernel(x, pos_embed, kf_embed, ln_weight, ln_bias, n_past, n_future,
           n_trans):
    b, s_len, d = x.shape
    bblk = 64
    chunk = 8
    scal = jnp.stack([jnp.asarray(n_past, jnp.int32),
                      jnp.asarray(n_future, jnp.int32),
                      jnp.asarray(n_trans, jnp.int32)])
    kf_pad = jnp.zeros((8, d), kf_embed.dtype).at[:3, :].set(kf_embed)

    return pl.pallas_call(
        functools.partial(_emb_ln_kernel, s_len=s_len, bblk=bblk, chunk=chunk),
        grid=(b // bblk,),
        in_specs=[
            pl.BlockSpec(memory_space=pltpu.SMEM),
            pl.BlockSpec((s_len, d), lambda i: (0, 0)),
            pl.BlockSpec((8, d), lambda i: (0, 0)),
            pl.BlockSpec((1, d), lambda i: (0, 0)),
            pl.BlockSpec((1, d), lambda i: (0, 0)),
            pl.BlockSpec((bblk, s_len, d), lambda i: (i, 0, 0)),
        ],
        out_specs=pl.BlockSpec((bblk, s_len, d), lambda i: (i, 0, 0)),
        out_shape=jax.ShapeDtypeStruct((b, s_len, d), x.dtype),
        compiler_params=pltpu.CompilerParams(
            dimension_semantics=("arbitrary",)),
    )(scal, pos_embed, kf_pad, ln_weight.reshape(1, d),
      ln_bias.reshape(1, d), x)


# speedup vs baseline: 1.0441x; 1.0441x over previous
import functools

import jax
import jax.numpy as jnp
from jax.experimental import pallas as pl
from jax.experimental.pallas import tpu as pltpu


def _emb_ln_kernel(scal_ref, pos_ref, kf_ref, w_ref, b_ref, x_ref, o_ref,
                   *, s_len, bblk, chunk):
    n_past = scal_ref[0]
    n_trans = scal_ref[2]
    n_position = n_past + scal_ref[1] + n_trans

    d = pos_ref.shape[1]
    s = jax.lax.broadcasted_iota(jnp.int32, (s_len, d), 0)
    in_trans = (s >= n_past) & (s < n_past + n_trans)
    beyond = s >= n_position
    kf_vec = jnp.where(beyond, kf_ref[2:3, :],
                       jnp.where(in_trans, kf_ref[1:2, :], kf_ref[0:1, :]))
    add = (pos_ref[...] + kf_vec)[None, :, :]
    w = w_ref[...]
    b = b_ref[...]
    inv_d = jnp.float32(1.0 / d)

    emb = x_ref[...] + add
    mean = jnp.sum(emb, axis=-1, keepdims=True) * inv_d
    msq = jnp.sum(emb * emb, axis=-1, keepdims=True) * inv_d
    var = msq - mean * mean
    scale = jax.lax.rsqrt(var + 1e-5)
    o_ref[...] = (emb - mean) * (scale * w) + b


def kernel(x, pos_embed, kf_embed, ln_weight, ln_bias, n_past, n_future,
           n_trans):
    b, s_len, d = x.shape
    bblk = 64
    chunk = 8
    scal = jnp.stack([jnp.asarray(n_past, jnp.int32),
                      jnp.asarray(n_future, jnp.int32),
                      jnp.asarray(n_trans, jnp.int32)])
    kf_pad = jnp.zeros((8, d), kf_embed.dtype).at[:3, :].set(kf_embed)

    return pl.pallas_call(
        functools.partial(_emb_ln_kernel, s_len=s_len, bblk=bblk, chunk=chunk),
        grid=(b // bblk,),
        in_specs=[
            pl.BlockSpec(memory_space=pltpu.SMEM),
            pl.BlockSpec((s_len, d), lambda i: (0, 0)),
            pl.BlockSpec((8, d), lambda i: (0, 0)),
            pl.BlockSpec((1, d), lambda i: (0, 0)),
            pl.BlockSpec((1, d), lambda i: (0, 0)),
            pl.BlockSpec((bblk, s_len, d), lambda i: (i, 0, 0)),
        ],
        out_specs=pl.BlockSpec((bblk, s_len, d), lambda i: (i, 0, 0)),
        out_shape=jax.ShapeDtypeStruct((b, s_len, d), x.dtype),
        compiler_params=pltpu.CompilerParams(
            dimension_semantics=("arbitrary",)),
    )(scal, pos_embed, kf_pad, ln_weight.reshape(1, d),
      ln_bias.reshape(1, d), x)


# speedup vs baseline: 1.2054x; 1.1545x over previous
import functools

import jax
import jax.numpy as jnp
from jax.experimental import pallas as pl
from jax.experimental.pallas import tpu as pltpu


def _emb_ln_kernel(scal_ref, pos_ref, kf_ref, w_ref, b_ref, x_ref, o_ref,
                   *, s_len, bblk, chunk):
    n_past = scal_ref[0]
    n_trans = scal_ref[2]
    n_position = n_past + scal_ref[1] + n_trans

    d = pos_ref.shape[1]
    s = jax.lax.broadcasted_iota(jnp.int32, (s_len, d), 0)
    in_trans = (s >= n_past) & (s < n_past + n_trans)
    beyond = s >= n_position
    kf_vec = jnp.where(beyond, kf_ref[2:3, :],
                       jnp.where(in_trans, kf_ref[1:2, :], kf_ref[0:1, :]))
    add = (pos_ref[...] + kf_vec)[None, :, :]
    w = w_ref[...]
    b = b_ref[...]
    inv_d = jnp.float32(1.0 / d)

    bblk = x_ref.shape[0]
    jmat = jnp.full((d, d), inv_d, dtype=jnp.float32)
    emb = (x_ref[...] + add).reshape(bblk * s_len, d)
    mean = jax.lax.dot(emb, jmat,
                       preferred_element_type=jnp.float32)
    diff = emb - mean
    var = jnp.sum(diff * diff, axis=-1, keepdims=True) * inv_d
    scale = jax.lax.rsqrt(var + 1e-5)
    o_ref[...] = (diff * (scale * w) + b).reshape(bblk, s_len, d)


def kernel(x, pos_embed, kf_embed, ln_weight, ln_bias, n_past, n_future,
           n_trans):
    b, s_len, d = x.shape
    bblk = 64
    chunk = 8
    scal = jnp.stack([jnp.asarray(n_past, jnp.int32),
                      jnp.asarray(n_future, jnp.int32),
                      jnp.asarray(n_trans, jnp.int32)])
    kf_pad = jnp.zeros((8, d), kf_embed.dtype).at[:3, :].set(kf_embed)

    return pl.pallas_call(
        functools.partial(_emb_ln_kernel, s_len=s_len, bblk=bblk, chunk=chunk),
        grid=(b // bblk,),
        in_specs=[
            pl.BlockSpec(memory_space=pltpu.SMEM),
            pl.BlockSpec((s_len, d), lambda i: (0, 0)),
            pl.BlockSpec((8, d), lambda i: (0, 0)),
            pl.BlockSpec((1, d), lambda i: (0, 0)),
            pl.BlockSpec((1, d), lambda i: (0, 0)),
            pl.BlockSpec((bblk, s_len, d), lambda i: (i, 0, 0)),
        ],
        out_specs=pl.BlockSpec((bblk, s_len, d), lambda i: (i, 0, 0)),
        out_shape=jax.ShapeDtypeStruct((b, s_len, d), x.dtype),
        compiler_params=pltpu.CompilerParams(
            dimension_semantics=("arbitrary",)),
    )(scal, pos_embed, kf_pad, ln_weight.reshape(1, d),
      ln_bias.reshape(1, d), x)
